# baseline (speedup 1.0000x reference)
import functools

import jax
import jax.numpy as jnp
from jax import lax
from jax.experimental import pallas as pl
from jax.experimental.pallas import tpu as pltpu
from jax.experimental.pallas import tpu_sc as plsc

NUM_EMBEDDINGS = 1000000
EMBEDDING_DIM = 64
BATCH = 16384
N_FIELDS = 26

NC = 2
NS = 16
NW = NC * NS

B_TOTAL = BATCH * N_FIELDS
B_PER_W = B_TOTAL // NW
CHUNK = 128
N_CHUNKS = B_PER_W // CHUNK
NBUF = 2


def _gather_body(ids_hbm, table_hbm, out_hbm,
                 idx_v, rows_v, in_sems, out_sems):
    wid = lax.axis_index("s") * NC + lax.axis_index("c")
    base = wid * B_PER_W

    def issue(g, buf):
        off = base + g * CHUNK
        pltpu.sync_copy(ids_hbm.at[pl.ds(off, CHUNK)], idx_v.at[buf])
        pltpu.async_copy(table_hbm.at[idx_v.at[buf]], rows_v.at[buf],
                         in_sems.at[buf])

    def drain(g, buf):
        pltpu.make_async_copy(table_hbm.at[idx_v.at[buf]], rows_v.at[buf],
                              in_sems.at[buf]).wait()
        off = base + g * CHUNK
        pltpu.async_copy(rows_v.at[buf], out_hbm.at[pl.ds(off, CHUNK)],
                         out_sems.at[buf])

    def wait_out(buf):
        pltpu.make_async_copy(rows_v.at[buf],
                              out_hbm.at[pl.ds(0, CHUNK)],
                              out_sems.at[buf]).wait()

    for b in range(NBUF):
        issue(b, b)

    def loop_body(g, _):
        for b in range(NBUF):
            drain(g + b, b)
            wait_out(b)
            issue(g + b + NBUF, b)
        return 0

    lax.fori_loop(0, (N_CHUNKS - NBUF) // NBUF, lambda i, c: loop_body(i * NBUF, c),
                  0, unroll=False)

    for b in range(NBUF):
        drain(N_CHUNKS - NBUF + b, b)
        wait_out(b)


@functools.partial(jax.jit, static_argnames=())
def _gather(ids_flat, table):
    mesh = plsc.VectorSubcoreMesh(core_axis_name="c", subcore_axis_name="s",
                                  num_cores=NC, num_subcores=NS)
    f = pl.kernel(
        _gather_body,
        out_type=jax.ShapeDtypeStruct((B_TOTAL, EMBEDDING_DIM), jnp.float32),
        mesh=mesh,
        scratch_types=[
            pltpu.VMEM((NBUF, CHUNK), jnp.int32),
            pltpu.VMEM((NBUF, CHUNK, EMBEDDING_DIM), jnp.float32),
            pltpu.SemaphoreType.DMA((NBUF,)),
            pltpu.SemaphoreType.DMA((NBUF,)),
        ],
        compiler_params=pltpu.CompilerParams(use_tc_tiling_on_sc=False),
    )
    return f(ids_flat, table)


def kernel(ids, table):
    ids_flat = ids.reshape(-1).astype(jnp.int32)
    out = _gather(ids_flat, table)
    return out.reshape(BATCH, N_FIELDS, EMBEDDING_DIM)


# speedup vs baseline: 1.0448x; 1.0448x over previous
import functools

import jax
import jax.numpy as jnp
from jax import lax
from jax.experimental import pallas as pl
from jax.experimental.pallas import tpu as pltpu
from jax.experimental.pallas import tpu_sc as plsc

NUM_EMBEDDINGS = 1000000
EMBEDDING_DIM = 64
BATCH = 16384
N_FIELDS = 26

NC = 2
NS = 16
NW = NC * NS

B_TOTAL = BATCH * N_FIELDS
B_PER_W = B_TOTAL // NW
CHUNK = 128
N_CHUNKS = B_PER_W // CHUNK
NBUF = 8
K = 4

N_ITERS = N_CHUNKS // NBUF


def _gather_body(ids_hbm, table_hbm, out_hbm,
                 idx_v, rows_v, in_sems, out_sems):
    wid = lax.axis_index("s") * NC + lax.axis_index("c")
    base = wid * B_PER_W

    pltpu.sync_copy(ids_hbm.at[wid], idx_v)

    def issue_gather(c, buf):
        pltpu.async_copy(table_hbm.at[idx_v.at[c]], rows_v.at[buf],
                         in_sems.at[buf])

    def wait_gather(c, buf):
        pltpu.make_async_copy(table_hbm.at[idx_v.at[c]], rows_v.at[buf],
                              in_sems.at[buf]).wait()

    def issue_store(c, buf):
        pltpu.async_copy(rows_v.at[buf], out_hbm.at[pl.ds(base + c * CHUNK, CHUNK)],
                         out_sems.at[buf])

    def wait_store(buf):
        pltpu.make_async_copy(rows_v.at[buf], out_hbm.at[pl.ds(base, CHUNK)],
                              out_sems.at[buf]).wait()

    for c in range(K):
        issue_gather(c, c % NBUF)

    def step(c, bi, *, first, last):
        nxt = (bi + K) % NBUF
        if last:
            pass
        else:
            if not (first and bi + K < NBUF):
                wait_store(nxt)
            issue_gather(c + K, nxt)
        wait_gather(c, bi)
        issue_store(c, bi)

    for bi in range(NBUF):
        step(bi, bi, first=True, last=False)

    def loop_body(i, _):
        g = i * NBUF
        for bi in range(NBUF):
            step(g + bi, bi, first=False, last=False)
        return 0

    lax.fori_loop(1, N_ITERS - 1, loop_body, 0, unroll=False)

    g = (N_ITERS - 1) * NBUF
    for bi in range(NBUF):
        step(g + bi, bi, first=False, last=(bi + K >= NBUF))

    for bi in range(NBUF):
        wait_store(bi)


@jax.jit
def _gather(ids_grouped, table):
    mesh = plsc.VectorSubcoreMesh(core_axis_name="c", subcore_axis_name="s",
                                  num_cores=NC, num_subcores=NS)
    f = pl.kernel(
        _gather_body,
        out_type=jax.ShapeDtypeStruct((B_TOTAL, EMBEDDING_DIM), jnp.float32),
        mesh=mesh,
        scratch_types=[
            pltpu.VMEM((N_CHUNKS, CHUNK), jnp.int32),
            pltpu.VMEM((NBUF, CHUNK, EMBEDDING_DIM), jnp.float32),
            pltpu.SemaphoreType.DMA((NBUF,)),
            pltpu.SemaphoreType.DMA((NBUF,)),
        ],
        compiler_params=pltpu.CompilerParams(use_tc_tiling_on_sc=False),
    )
    return f(ids_grouped, table)


def kernel(ids, table):
    ids_grouped = ids.reshape(NW, N_CHUNKS, CHUNK).astype(jnp.int32)
    out = _gather(ids_grouped, table)
    return out.reshape(BATCH, N_FIELDS, EMBEDDING_DIM)
